# baseline (speedup 1.0000x reference)
import functools

import jax
import jax.numpy as jnp
from jax import lax
from jax.experimental import pallas as pl
from jax.experimental.pallas import tpu as pltpu
from jax.experimental.pallas import tpu_sc as plsc

NC = 2
NS = 16
NT = NC * NS
NPT = 320
NPAD = NT * NPT
BLK = 128
DCH = 16
NUM_ITERS = 40


def _step_kernel(d_feat, n_iters):
    grp = BLK // 16

    mesh = plsc.VectorSubcoreMesh(core_axis_name="c", subcore_axis_name="s")

    @functools.partial(
        pl.kernel,
        out_type=jax.ShapeDtypeStruct((NPAD, d_feat), jnp.float32),
        mesh=mesh,
        scratch_types=[
            pltpu.VMEM((NPT, d_feat), jnp.float32),
            pltpu.VMEM((BLK, d_feat), jnp.float32),
            pltpu.VMEM((BLK,), jnp.int32),
            pltpu.VMEM((BLK,), jnp.float32),
            pltpu.VMEM((BLK,), jnp.int32),
            pltpu.VMEM((NT,), jnp.int32),
            pltpu.VMEM((NT,), jnp.int32),
            pltpu.SemaphoreType.DMA,
        ],
        compiler_params=pltpu.CompilerParams(needs_layout_passes=False),
    )
    def step(out_in, xm, colp, dadp, dstp, offs, nblk,
             out_next, acc, gbuf, colv, dadv, dstv, offs_v, nblk_v, sem):
        wid = lax.axis_index("s") * NC + lax.axis_index("c")
        base = wid * NPT
        pltpu.sync_copy(offs, offs_v)
        pltpu.sync_copy(nblk, nblk_v)
        pltpu.sync_copy(xm.at[pl.ds(base, NPT)], acc)
        widv = jnp.full((16,), wid, jnp.int32)
        off = pl.multiple_of(jnp.max(plsc.load_gather(offs_v, [widv])), BLK)
        nb = jnp.max(plsc.load_gather(nblk_v, [widv]))
        rows_g = [lax.iota(jnp.int32, 16) + 16 * g for g in range(grp)]

        def block_body(b, carry):
            ebase = pl.multiple_of(off + b * BLK, BLK)
            pltpu.sync_copy(colp.at[pl.ds(ebase, BLK)], colv)
            pltpu.sync_copy(dadp.at[pl.ds(ebase, BLK)], dadv)
            pltpu.sync_copy(dstp.at[pl.ds(ebase, BLK)], dstv)
            pltpu.async_copy(out_in.at[colv], gbuf, sem).wait()
            dadg = [dadv[pl.ds(16 * g, 16)] for g in range(grp)]
            dstg = [dstv[pl.ds(16 * g, 16)] for g in range(grp)]

            def d_body(dc, inner):
                for dd in range(DCH):
                    d = dc * DCH + dd
                    colidx = jnp.full((16,), d, jnp.int32)
                    for g in range(grp):
                        vals = plsc.load_gather(gbuf, [rows_g[g], colidx])
                        plsc.addupdate_scatter(
                            acc, [dstg[g], colidx], vals * dadg[g])
                return inner

            lax.fori_loop(0, d_feat // DCH, d_body, 0)
            return carry

        lax.fori_loop(0, nb, block_body, 0)
        pltpu.sync_copy(acc, out_next.at[pl.ds(base, NPT)])

    return step


def kernel(x, edge_index, mask):
    n, d = x.shape
    e = edge_index.shape[1]
    f32, i32 = jnp.float32, jnp.int32

    dst = edge_index[0]
    src = edge_index[1]

    deg = jnp.zeros((n,), f32).at[src].add(jnp.ones((e,), f32))
    dinv = jnp.where(deg > 0, lax.rsqrt(jnp.maximum(deg, 1.0)), 0.0)
    dad = dinv[dst] * dinv[src] * (1.0 - mask[dst].astype(f32))

    xm = jnp.where(mask[:, None], x, 0.0)
    xm_pad = jnp.zeros((NPAD, d), f32).at[:n].set(xm)

    perm1 = jnp.argsort(dst)
    dst_s = dst[perm1]
    idx = jnp.arange(e, dtype=i32)
    is_start = jnp.concatenate(
        [jnp.ones((1,), jnp.bool_), dst_s[1:] != dst_s[:-1]])
    seg_start = lax.associative_scan(jnp.maximum, jnp.where(is_start, idx, 0))
    rank = idx - seg_start
    tile_s = dst_s // NPT
    key = tile_s * (1 << 19) + jnp.minimum(rank, (1 << 19) - 1)
    perm = perm1[jnp.argsort(key)]

    dst_p = dst[perm]
    tile_p = dst_p // NPT
    counts = jnp.zeros((NT,), i32).at[tile_p].add(1)
    padded = ((counts + BLK - 1) // BLK) * BLK
    offs = jnp.concatenate([jnp.zeros((1,), i32), jnp.cumsum(padded)])[:NT]
    nblk = padded // BLK
    excl = jnp.concatenate([jnp.zeros((1,), i32), jnp.cumsum(counts)])[tile_p]
    pos = offs[tile_p] + (idx - excl)

    p_tot = e + NT * BLK
    colp = jnp.zeros((p_tot,), i32).at[pos].set(src[perm])
    dadp = jnp.zeros((p_tot,), f32).at[pos].set(dad[perm])
    dstp = jnp.zeros((p_tot,), i32).at[pos].set(dst_p - tile_p * NPT)

    step = _step_kernel(d, NUM_ITERS)
    out = xm_pad
    for _ in range(NUM_ITERS):
        out = step(out, xm_pad, colp, dadp, dstp, offs, nblk)
    return out[:n]


# speedup vs baseline: 1.1011x; 1.1011x over previous
import functools

import jax
import jax.numpy as jnp
from jax import lax
from jax.experimental import pallas as pl
from jax.experimental.pallas import tpu as pltpu
from jax.experimental.pallas import tpu_sc as plsc

NC = 2
NS = 16
NT = NC * NS
NPT = 320
NPAD = NT * NPT
BLK = 128
DCH = 16
NUM_ITERS = 40


def _step_kernel(d_feat, n_iters):
    grp = BLK // 16

    mesh = plsc.VectorSubcoreMesh(core_axis_name="c", subcore_axis_name="s")

    @functools.partial(
        pl.kernel,
        out_type=jax.ShapeDtypeStruct((NPAD, d_feat), jnp.float32),
        mesh=mesh,
        scratch_types=[
            pltpu.VMEM((NPT, d_feat), jnp.float32),
            pltpu.VMEM((BLK, d_feat), jnp.float32),
            pltpu.VMEM((BLK, d_feat), jnp.float32),
            pltpu.VMEM((BLK,), jnp.int32),
            pltpu.VMEM((BLK,), jnp.int32),
            pltpu.VMEM((BLK,), jnp.float32),
            pltpu.VMEM((BLK,), jnp.float32),
            pltpu.VMEM((BLK,), jnp.int32),
            pltpu.VMEM((BLK,), jnp.int32),
            pltpu.VMEM((NT,), jnp.int32),
            pltpu.VMEM((NT,), jnp.int32),
            pltpu.SemaphoreType.DMA,
            pltpu.SemaphoreType.DMA,
            pltpu.SemaphoreType.DMA,
            pltpu.SemaphoreType.DMA,
            pltpu.SemaphoreType.DMA,
            pltpu.SemaphoreType.DMA,
            pltpu.SemaphoreType.DMA,
            pltpu.SemaphoreType.DMA,
        ],
        compiler_params=pltpu.CompilerParams(needs_layout_passes=False),
    )
    def step(out_in, xm, colp, dadp, dstp, offs, nblk,
             out_next, acc, gbuf0, gbuf1, colv0, colv1, dadv0, dadv1,
             dstv0, dstv1, offs_v, nblk_v,
             gsem0, gsem1, csem0, csem1, asem0, asem1, dsem0, dsem1):
        gbuf = [gbuf0, gbuf1]
        colv = [colv0, colv1]
        dadv = [dadv0, dadv1]
        dstv = [dstv0, dstv1]
        gsem = [gsem0, gsem1]
        csem = [csem0, csem1]
        asem = [asem0, asem1]
        dsem = [dsem0, dsem1]

        wid = lax.axis_index("s") * NC + lax.axis_index("c")
        base = wid * NPT
        pltpu.sync_copy(offs, offs_v)
        pltpu.sync_copy(nblk, nblk_v)
        pltpu.sync_copy(xm.at[pl.ds(base, NPT)], acc)
        widv = jnp.full((16,), wid, jnp.int32)
        off = pl.multiple_of(jnp.max(plsc.load_gather(offs_v, [widv])), BLK)
        nb = jnp.max(plsc.load_gather(nblk_v, [widv]))
        rows_g = [lax.iota(jnp.int32, 16) + 16 * g for g in range(grp)]

        def eslice(ref, b):
            return ref.at[pl.ds(pl.multiple_of(off + b * BLK, BLK), BLK)]

        def start_col(b, j):
            pltpu.make_async_copy(eslice(colp, b), colv[j], csem[j]).start()

        def start_daddst(b, j):
            pltpu.make_async_copy(eslice(dadp, b), dadv[j], asem[j]).start()
            pltpu.make_async_copy(eslice(dstp, b), dstv[j], dsem[j]).start()

        def start_gather(j):
            pltpu.make_async_copy(out_in.at[colv[j]], gbuf[j], gsem[j]).start()

        def compute(j):
            dadg = [dadv[j][pl.ds(16 * g, 16)] for g in range(grp)]
            dstg = [dstv[j][pl.ds(16 * g, 16)] for g in range(grp)]

            def d_body(dc, inner):
                for dd in range(DCH):
                    d = dc * DCH + dd
                    colidx = jnp.full((16,), d, jnp.int32)
                    for g in range(grp):
                        vals = plsc.load_gather(gbuf[j], [rows_g[g], colidx])
                        plsc.addupdate_scatter(
                            acc, [dstg[g], colidx], vals * dadg[g])
                return inner

            lax.fori_loop(0, d_feat // DCH, d_body, 0)

        @pl.when(nb > 0)
        def _():
            start_col(0, 0)
            start_daddst(0, 0)
        @pl.when(nb > 1)
        def _():
            start_col(1, 1)
        @pl.when(nb > 0)
        def _():
            pltpu.make_async_copy(eslice(colp, 0), colv[0], csem[0]).wait()
            start_gather(0)

        def pair_body(p, carry):
            for j in range(2):
                b = 2 * p + j
                jn = 1 - j

                @pl.when(b + 1 < nb)
                def _():
                    pltpu.make_async_copy(
                        eslice(colp, b + 1), colv[jn], csem[jn]).wait()
                    start_gather(jn)
                    start_daddst(b + 1, jn)

                @pl.when(b + 2 < nb)
                def _():
                    start_col(b + 2, j)

                @pl.when(b < nb)
                def _():
                    pltpu.make_async_copy(
                        out_in.at[colv[j]], gbuf[j], gsem[j]).wait()
                    pltpu.make_async_copy(
                        eslice(dadp, b), dadv[j], asem[j]).wait()
                    pltpu.make_async_copy(
                        eslice(dstp, b), dstv[j], dsem[j]).wait()
                    compute(j)
            return carry

        lax.fori_loop(0, (nb + 1) // 2, pair_body, 0)
        pltpu.sync_copy(acc, out_next.at[pl.ds(base, NPT)])

    return step


def kernel(x, edge_index, mask):
    n, d = x.shape
    e = edge_index.shape[1]
    f32, i32 = jnp.float32, jnp.int32

    dst = edge_index[0]
    src = edge_index[1]

    deg = jnp.zeros((n,), f32).at[src].add(jnp.ones((e,), f32))
    dinv = jnp.where(deg > 0, lax.rsqrt(jnp.maximum(deg, 1.0)), 0.0)
    dad = dinv[dst] * dinv[src] * (1.0 - mask[dst].astype(f32))

    xm = jnp.where(mask[:, None], x, 0.0)
    xm_pad = jnp.zeros((NPAD, d), f32).at[:n].set(xm)

    perm1 = jnp.argsort(dst)
    dst_s = dst[perm1]
    idx = jnp.arange(e, dtype=i32)
    is_start = jnp.concatenate(
        [jnp.ones((1,), jnp.bool_), dst_s[1:] != dst_s[:-1]])
    seg_start = lax.associative_scan(jnp.maximum, jnp.where(is_start, idx, 0))
    rank = idx - seg_start
    tile_s = dst_s // NPT
    key = tile_s * (1 << 19) + jnp.minimum(rank, (1 << 19) - 1)
    perm = perm1[jnp.argsort(key)]

    dst_p = dst[perm]
    tile_p = dst_p // NPT
    counts = jnp.zeros((NT,), i32).at[tile_p].add(1)
    padded = ((counts + BLK - 1) // BLK) * BLK
    offs = jnp.concatenate([jnp.zeros((1,), i32), jnp.cumsum(padded)])[:NT]
    nblk = padded // BLK
    excl = jnp.concatenate([jnp.zeros((1,), i32), jnp.cumsum(counts)])[tile_p]
    pos = offs[tile_p] + (idx - excl)

    p_tot = e + NT * BLK
    colp = jnp.zeros((p_tot,), i32).at[pos].set(src[perm])
    dadp = jnp.zeros((p_tot,), f32).at[pos].set(dad[perm])
    dstp = jnp.zeros((p_tot,), i32).at[pos].set(dst_p - tile_p * NPT)

    step = _step_kernel(d, NUM_ITERS)
    out = xm_pad
    for _ in range(NUM_ITERS):
        out = step(out, xm_pad, colp, dadp, dstp, offs, nblk)
    return out[:n]


# speedup vs baseline: 2.9559x; 2.6845x over previous
import functools

import jax
import jax.numpy as jnp
from jax import lax
from jax.experimental import pallas as pl
from jax.experimental.pallas import tpu as pltpu
from jax.experimental.pallas import tpu_sc as plsc

NC = 2
NS = 16
NT = NC * NS
NPT = 320
NPAD = NT * NPT
BLK = 128
DCH = 16
NUM_ITERS = 40


def _step_kernel(d_feat, n_iters):
    grp = BLK // 16

    mesh = plsc.VectorSubcoreMesh(core_axis_name="c", subcore_axis_name="s")

    @functools.partial(
        pl.kernel,
        out_type=jax.ShapeDtypeStruct((NPAD, d_feat), jnp.float32),
        mesh=mesh,
        scratch_types=[
            pltpu.VMEM((NPT, d_feat), jnp.float32),
            pltpu.VMEM((BLK, d_feat), jnp.float32),
            pltpu.VMEM((BLK, d_feat), jnp.float32),
            pltpu.VMEM((BLK,), jnp.int32),
            pltpu.VMEM((BLK,), jnp.int32),
            pltpu.VMEM((BLK,), jnp.float32),
            pltpu.VMEM((BLK,), jnp.float32),
            pltpu.VMEM((BLK,), jnp.int32),
            pltpu.VMEM((BLK,), jnp.int32),
            pltpu.VMEM((NT,), jnp.int32),
            pltpu.VMEM((NT,), jnp.int32),
            pltpu.SemaphoreType.DMA,
            pltpu.SemaphoreType.DMA,
            pltpu.SemaphoreType.DMA,
            pltpu.SemaphoreType.DMA,
            pltpu.SemaphoreType.DMA,
            pltpu.SemaphoreType.DMA,
            pltpu.SemaphoreType.DMA,
            pltpu.SemaphoreType.DMA,
        ],
        compiler_params=pltpu.CompilerParams(needs_layout_passes=False),
    )
    def step(out_in, xm, colp, dadp, dstp, offs, nblk,
             out_next, acc, gbuf0, gbuf1, colv0, colv1, dadv0, dadv1,
             dstv0, dstv1, offs_v, nblk_v,
             gsem0, gsem1, csem0, csem1, asem0, asem1, dsem0, dsem1):
        gbuf = [gbuf0, gbuf1]
        colv = [colv0, colv1]
        dadv = [dadv0, dadv1]
        dstv = [dstv0, dstv1]
        gsem = [gsem0, gsem1]
        csem = [csem0, csem1]
        asem = [asem0, asem1]
        dsem = [dsem0, dsem1]

        wid = lax.axis_index("s") * NC + lax.axis_index("c")
        base = wid * NPT
        pltpu.sync_copy(offs, offs_v)
        pltpu.sync_copy(nblk, nblk_v)
        pltpu.sync_copy(xm.at[pl.ds(base, NPT)], acc)
        widv = jnp.full((16,), wid, jnp.int32)
        off = pl.multiple_of(jnp.max(plsc.load_gather(offs_v, [widv])), BLK)
        nb = jnp.max(plsc.load_gather(nblk_v, [widv]))
        rows_g = [lax.iota(jnp.int32, 16) + 16 * g for g in range(grp)]

        def eslice(ref, b):
            return ref.at[pl.ds(pl.multiple_of(off + b * BLK, BLK), BLK)]

        def start_col(b, j):
            pltpu.make_async_copy(eslice(colp, b), colv[j], csem[j]).start()

        def start_daddst(b, j):
            pltpu.make_async_copy(eslice(dadp, b), dadv[j], asem[j]).start()
            pltpu.make_async_copy(eslice(dstp, b), dstv[j], dsem[j]).start()

        def start_gather(j):
            pltpu.make_async_copy(out_in.at[colv[j]], gbuf[j], gsem[j]).start()

        lanes = lax.iota(jnp.int32, 16)

        def compute(j):
            dadg = [dadv[j][pl.ds(16 * g, 16)] for g in range(grp)]
            dstg = [dstv[j][pl.ds(16 * g, 16)] for g in range(grp)]

            def d_body(w, inner):
                for s in range(DCH):
                    colidx = w * 16 + ((s + lanes) & 15)
                    for g in range(grp):
                        vals = plsc.load_gather(gbuf[j], [rows_g[g], colidx])
                        plsc.addupdate_scatter(
                            acc, [dstg[g], colidx], vals * dadg[g])
                return inner

            lax.fori_loop(0, d_feat // DCH, d_body, 0)

        @pl.when(nb > 0)
        def _():
            start_col(0, 0)
            start_daddst(0, 0)
        @pl.when(nb > 1)
        def _():
            start_col(1, 1)
        @pl.when(nb > 0)
        def _():
            pltpu.make_async_copy(eslice(colp, 0), colv[0], csem[0]).wait()
            start_gather(0)

        def pair_body(p, carry):
            for j in range(2):
                b = 2 * p + j
                jn = 1 - j

                @pl.when(b + 1 < nb)
                def _():
                    pltpu.make_async_copy(
                        eslice(colp, b + 1), colv[jn], csem[jn]).wait()
                    start_gather(jn)
                    start_daddst(b + 1, jn)

                @pl.when(b + 2 < nb)
                def _():
                    start_col(b + 2, j)

                @pl.when(b < nb)
                def _():
                    pltpu.make_async_copy(
                        out_in.at[colv[j]], gbuf[j], gsem[j]).wait()
                    pltpu.make_async_copy(
                        eslice(dadp, b), dadv[j], asem[j]).wait()
                    pltpu.make_async_copy(
                        eslice(dstp, b), dstv[j], dsem[j]).wait()
                    compute(j)
            return carry

        lax.fori_loop(0, (nb + 1) // 2, pair_body, 0)
        pltpu.sync_copy(acc, out_next.at[pl.ds(base, NPT)])

    return step


def kernel(x, edge_index, mask):
    n, d = x.shape
    e = edge_index.shape[1]
    f32, i32 = jnp.float32, jnp.int32

    dst = edge_index[0]
    src = edge_index[1]

    deg = jnp.zeros((n,), f32).at[src].add(jnp.ones((e,), f32))
    dinv = jnp.where(deg > 0, lax.rsqrt(jnp.maximum(deg, 1.0)), 0.0)
    dad = dinv[dst] * dinv[src] * (1.0 - mask[dst].astype(f32))

    xm = jnp.where(mask[:, None], x, 0.0)
    xm_pad = jnp.zeros((NPAD, d), f32).at[:n].set(xm)

    perm1 = jnp.argsort(dst)
    dst_s = dst[perm1]
    idx = jnp.arange(e, dtype=i32)
    is_start = jnp.concatenate(
        [jnp.ones((1,), jnp.bool_), dst_s[1:] != dst_s[:-1]])
    seg_start = lax.associative_scan(jnp.maximum, jnp.where(is_start, idx, 0))
    rank = idx - seg_start
    tile_s = dst_s // NPT
    key = tile_s * (1 << 19) + jnp.minimum(rank, (1 << 19) - 1)
    perm = perm1[jnp.argsort(key)]

    dst_p = dst[perm]
    tile_p = dst_p // NPT
    counts = jnp.zeros((NT,), i32).at[tile_p].add(1)
    padded = ((counts + BLK - 1) // BLK) * BLK
    offs = jnp.concatenate([jnp.zeros((1,), i32), jnp.cumsum(padded)])[:NT]
    nblk = padded // BLK
    excl = jnp.concatenate([jnp.zeros((1,), i32), jnp.cumsum(counts)])[tile_p]
    pos = offs[tile_p] + (idx - excl)

    p_tot = e + NT * BLK
    colp = jnp.zeros((p_tot,), i32).at[pos].set(src[perm])
    dadp = jnp.zeros((p_tot,), f32).at[pos].set(dad[perm])
    dstp = jnp.zeros((p_tot,), i32).at[pos].set(dst_p - tile_p * NPT)

    step = _step_kernel(d, NUM_ITERS)
    out = xm_pad
    for _ in range(NUM_ITERS):
        out = step(out, xm_pad, colp, dadp, dstp, offs, nblk)
    return out[:n]


# speedup vs baseline: 6.5329x; 2.2101x over previous
import functools

import jax
import jax.numpy as jnp
from jax import lax
from jax.experimental import pallas as pl
from jax.experimental.pallas import tpu as pltpu
from jax.experimental.pallas import tpu_sc as plsc

NC = 2
NS = 16
NT = NC * NS
NPT = 320
NPAD = NT * NPT
SUB = 2
SB = SUB * 128
NUM_ITERS = 40


def _step_kernel(d_feat):
    mesh = plsc.VectorSubcoreMesh(core_axis_name="c", subcore_axis_name="s")

    @functools.partial(
        pl.kernel,
        out_type=jax.ShapeDtypeStruct((NPAD, d_feat), jnp.float32),
        mesh=mesh,
        scratch_types=[
            pltpu.VMEM_SHARED((NS, NPT, d_feat), jnp.float32),
            pltpu.VMEM((64, d_feat), jnp.float32),
            pltpu.VMEM((SB, d_feat), jnp.float32),
            pltpu.VMEM((SB, d_feat), jnp.float32),
            pltpu.VMEM((SUB, 128), jnp.int32),
            pltpu.VMEM((SUB, 128), jnp.int32),
            pltpu.VMEM((SUB, 128), jnp.int32),
            pltpu.VMEM((SUB, 128), jnp.int32),
            pltpu.VMEM((NPT,), jnp.float32),
            pltpu.VMEM((NT,), jnp.int32),
            pltpu.VMEM((NT,), jnp.int32),
            pltpu.SemaphoreType.DMA,
            pltpu.SemaphoreType.DMA,
            pltpu.SemaphoreType.DMA,
            pltpu.SemaphoreType.DMA,
            pltpu.SemaphoreType.DMA,
            pltpu.SemaphoreType.DMA,
            pltpu.SemaphoreType.DMA,
            pltpu.SemaphoreType.DMA,
        ],
        compiler_params=pltpu.CompilerParams(needs_layout_passes=False),
    )
    def step(outs, zrows, parr, qarr, colb, dstb, offsb, nsb,
             out_next, acc_sh, accl, gbuf0, gbuf1, colv0, colv1, dstv0,
             dstv1, qv, offs_v, nsb_v,
             gsem0, gsem1, csem0, csem1, dsem0, dsem1, ssem0, ssem1):
        gbuf = [gbuf0, gbuf1]
        colv = [colv0, colv1]
        dstv = [dstv0, dstv1]
        gsem = [gsem0, gsem1]
        csem = [csem0, csem1]
        dsem = [dsem0, dsem1]
        ssem = [ssem0, ssem1]

        sid = lax.axis_index("s")
        wid = sid * NC + lax.axis_index("c")
        base = wid * NPT
        acc = acc_sh.at[sid]
        pltpu.sync_copy(offsb, offs_v)
        pltpu.sync_copy(nsb, nsb_v)
        pltpu.sync_copy(zrows, acc)
        pltpu.sync_copy(qarr.at[pl.ds(pl.multiple_of(base, 8), NPT)], qv)
        widv = jnp.full((16,), wid, jnp.int32)
        off = jnp.max(plsc.load_gather(offs_v, [widv]))
        nb = jnp.max(plsc.load_gather(nsb_v, [widv]))

        def ecopy(hbm, b, vbuf, sem):
            return pltpu.make_async_copy(
                hbm.at[pl.ds((off + b) * SUB, SUB)], vbuf, sem)

        def gsub(j, s):
            return pltpu.make_async_copy(
                outs.at[colv[j].at[s]], gbuf[j].at[pl.ds(s * 128, 128)],
                gsem[j])

        def start_gather(j):
            for s in range(SUB):
                pltpu.async_copy(
                    outs.at[colv[j].at[s]], gbuf[j].at[pl.ds(s * 128, 128)],
                    gsem[j])

        def ssub(j, s):
            return pltpu.make_async_copy(
                gbuf[j].at[pl.ds(s * 128, 128)], acc.at[dstv[j].at[s]],
                ssem[j])

        def start_scatter(j):
            pltpu.async_copy(
                gbuf[j].at[pl.ds(0, 128)], acc.at[dstv[j].at[0]],
                ssem[j], add=True)
            ssub(j, 0).wait()
            pltpu.async_copy(
                gbuf[j].at[pl.ds(128, 128)], acc.at[dstv[j].at[1]],
                ssem[j], add=True)

        def wait_scatter(j):
            ssub(j, 1).wait()

        @pl.when(nb > 0)
        def _():
            ecopy(colb, 0, colv[0], csem[0]).start()
            ecopy(dstb, 0, dstv[0], dsem[0]).start()
        @pl.when(nb > 1)
        def _():
            ecopy(colb, 1, colv[1], csem[1]).start()
        @pl.when(nb > 0)
        def _():
            ecopy(colb, 0, colv[0], csem[0]).wait()
            start_gather(0)

        def pair_body(p, carry):
            for j in range(2):
                b = 2 * p + j
                jn = 1 - j

                @pl.when(jnp.logical_and(b + 1 < nb, b >= 1))
                def _():
                    wait_scatter(jn)

                @pl.when(b + 1 < nb)
                def _():
                    ecopy(colb, b + 1, colv[jn], csem[jn]).wait()
                    start_gather(jn)
                    ecopy(dstb, b + 1, dstv[jn], dsem[jn]).start()

                @pl.when(b < nb)
                def _():
                    for s in range(SUB):
                        gsub(j, s).wait()
                    ecopy(dstb, b, dstv[j], dsem[j]).wait()
                    start_scatter(j)

                @pl.when(b + 2 < nb)
                def _():
                    ecopy(colb, b + 2, colv[j], csem[j]).start()
            return carry

        lax.fori_loop(0, (nb + 1) // 2, pair_body, 0)

        for q in range(2):
            @pl.when(jnp.logical_and(nb >= 1, (nb - 1) % 2 == q))
            def _():
                wait_scatter(q)
            @pl.when(jnp.logical_and(nb >= 2, (nb - 2) % 2 == q))
            def _():
                wait_scatter(q)

        lanes = lax.iota(jnp.int32, 16)
        for c in range(NPT // 64):
            pltpu.sync_copy(acc.at[pl.ds(c * 64, 64)], accl)
            pltpu.sync_copy(
                parr.at[pl.ds(pl.multiple_of(base + c * 64, 8), 64)],
                gbuf1.at[pl.ds(0, 64)])
            for rg in range(4):
                rowsv = jnp.full((16,), rg * 16, jnp.int32) + lanes
                qv16 = qv[pl.ds(c * 64 + rg * 16, 16)]

                def w_body(w, carry, rowsv=rowsv, qv16=qv16):
                    for s in range(16):
                        colidx = w * 16 + ((s + lanes) & 15)
                        av = plsc.load_gather(accl, [rowsv, colidx])
                        pv = plsc.load_gather(gbuf1, [rowsv, colidx])
                        plsc.store_scatter(accl, [rowsv, colidx],
                                           pv + qv16 * av)
                    return carry

                lax.fori_loop(0, d_feat // 16, w_body, 0)

            pltpu.sync_copy(
                accl, out_next.at[pl.ds(pl.multiple_of(base + c * 64, 8),
                                        64)])

    return step


def kernel(x, edge_index, mask):
    n, d = x.shape
    e = edge_index.shape[1]
    f32, i32 = jnp.float32, jnp.int32

    dst = edge_index[0]
    src = edge_index[1]

    deg = jnp.zeros((n,), f32).at[src].add(jnp.ones((e,), f32))
    dinv = jnp.where(deg > 0, lax.rsqrt(jnp.maximum(deg, 1.0)), 0.0)
    a = dinv * (1.0 - mask.astype(f32))

    xm = jnp.where(mask[:, None], x, 0.0)
    xm_pad = jnp.zeros((NPAD, d), f32).at[:n].set(xm)
    dinv_pad = jnp.zeros((NPAD,), f32).at[:n].set(dinv)
    a_pad = jnp.zeros((NPAD,), f32).at[:n].set(a)

    s0 = dinv_pad[:, None] * xm_pad
    pv = s0
    qmid = dinv_pad * a_pad

    perm1 = jnp.argsort(dst)
    idx = jnp.arange(e, dtype=i32)
    dst_p = dst[perm1]
    tile_p = dst_p // NPT
    counts = jnp.zeros((NT,), i32).at[tile_p].add(1)
    padded = ((counts + SB - 1) // SB) * SB
    offs_e = jnp.concatenate([jnp.zeros((1,), i32), jnp.cumsum(padded)])[:NT]
    nsb = padded // SB
    excl = jnp.concatenate([jnp.zeros((1,), i32), jnp.cumsum(counts)])[tile_p]
    pos = offs_e[tile_p] + (idx - excl)

    p_tot = e + NT * SB
    colp = jnp.full((p_tot,), NPAD - 1, i32).at[pos].set(src[perm1])
    dstp = jnp.zeros((p_tot,), i32).at[pos].set(dst_p - tile_p * NPT)

    colb = colp.reshape(p_tot // 128, 128)
    dstb = dstp.reshape(p_tot // 128, 128)
    offsb = offs_e // SB
    zrows = jnp.zeros((NPT, d), f32)

    step = _step_kernel(d)
    out = s0
    for it in range(NUM_ITERS):
        if it < NUM_ITERS - 1:
            out = step(out, zrows, pv, qmid, colb, dstb, offsb, nsb)
        else:
            out = step(out, zrows, xm_pad, a_pad, colb, dstb, offsb, nsb)
    return out[:n]
